# baseline (speedup 1.0000x reference)
import functools

import jax
import jax.numpy as jnp
from jax import lax
from jax.experimental import pallas as pl
from jax.experimental.pallas import tpu as pltpu

LANE = 128
_EPS = 1e-5


def _conv_kernel(x_ref, w_ref, scale_ref, shift_ref,
                 y_ref, sum_ref, sq_ref, xf_ref, pat_ref,
                 *, bn_relu_in, H, W):
    C = x_ref.shape[1]
    P = H * W
    base = W + 1

    x = x_ref[0]
    if bn_relu_in:
        s = scale_ref[...]
        t = shift_ref[...]
        x = jnp.maximum(x.astype(jnp.float32) * s + t, 0.0)
    xb = x.astype(jnp.bfloat16)

    xf_ref[...] = jnp.zeros_like(xf_ref)
    xf_ref[:, base:base + P] = xb

    j = lax.rem(lax.broadcasted_iota(jnp.int32, (1, P), 1), W)
    mask_l = (j != 0).astype(jnp.bfloat16)
    mask_r = (j != W - 1).astype(jnp.bfloat16)

    for kh in range(3):
        for kw in range(3):
            t_idx = kh * 3 + kw
            b = base + (kh - 1) * W + (kw - 1)
            p = xf_ref[:, b:b + P]
            if kw == 0:
                p = p * mask_l
            elif kw == 2:
                p = p * mask_r
            pat_ref[t_idx * C:(t_idx + 1) * C, :] = p

    acc = jnp.dot(w_ref[...], pat_ref[...],
                  preferred_element_type=jnp.float32)

    sum_ref[0] = jnp.sum(acc, axis=1, keepdims=True)
    sq_ref[0] = jnp.sum(acc * acc, axis=1, keepdims=True)

    y_ref[0] = acc.astype(y_ref.dtype)


def _bn_relu_kernel(y_ref, scale_ref, shift_ref, o_ref):
    s = scale_ref[...]
    t = shift_ref[...]
    o_ref[0] = jnp.maximum(y_ref[0].astype(jnp.float32) * s + t, 0.0
                           ).astype(o_ref.dtype)


def _conv_stage(x, w_packed, in_scale, in_shift, *, bn_relu_in, H, W):
    B, C, P = x.shape
    XF = P + 2 * W + 2
    XF = (XF + LANE - 1) // LANE * LANE

    kern = functools.partial(_conv_kernel, bn_relu_in=bn_relu_in, H=H, W=W)
    grid_spec = pltpu.PrefetchScalarGridSpec(
        num_scalar_prefetch=0,
        grid=(B,),
        in_specs=[
            pl.BlockSpec((1, C, P), lambda b: (b, 0, 0)),
            pl.BlockSpec((C, 9 * C), lambda b: (0, 0)),
            pl.BlockSpec((C, 1), lambda b: (0, 0)),
            pl.BlockSpec((C, 1), lambda b: (0, 0)),
        ],
        out_specs=[
            pl.BlockSpec((1, C, P), lambda b: (b, 0, 0)),
            pl.BlockSpec((1, C, 1), lambda b: (b, 0, 0)),
            pl.BlockSpec((1, C, 1), lambda b: (b, 0, 0)),
        ],
        scratch_shapes=[
            pltpu.VMEM((C, XF), jnp.bfloat16),
            pltpu.VMEM((9 * C, P), jnp.bfloat16),
        ],
    )
    y, s, sq = pl.pallas_call(
        kern,
        out_shape=(
            jax.ShapeDtypeStruct((B, C, P), jnp.bfloat16),
            jax.ShapeDtypeStruct((B, C, 1), jnp.float32),
            jax.ShapeDtypeStruct((B, C, 1), jnp.float32),
        ),
        grid_spec=grid_spec,
        compiler_params=pltpu.CompilerParams(
            dimension_semantics=("parallel",),
            vmem_limit_bytes=64 * 1024 * 1024,
        ),
    )(x, w_packed, in_scale, in_shift)
    return y, s, sq


def _bn_relu(y, scale, shift, out_dtype):
    B, C, P = y.shape
    grid_spec = pltpu.PrefetchScalarGridSpec(
        num_scalar_prefetch=0,
        grid=(B,),
        in_specs=[
            pl.BlockSpec((1, C, P), lambda b: (b, 0, 0)),
            pl.BlockSpec((C, 1), lambda b: (0, 0)),
            pl.BlockSpec((C, 1), lambda b: (0, 0)),
        ],
        out_specs=pl.BlockSpec((1, C, P), lambda b: (b, 0, 0)),
    )
    return pl.pallas_call(
        _bn_relu_kernel,
        out_shape=jax.ShapeDtypeStruct((B, C, P), out_dtype),
        grid_spec=grid_spec,
        compiler_params=pltpu.CompilerParams(
            dimension_semantics=("parallel",),
            vmem_limit_bytes=64 * 1024 * 1024,
        ),
    )(y, scale, shift)


def _pack_w(w_hwio):
    co = w_hwio.shape[3]
    return jnp.transpose(w_hwio, (3, 0, 1, 2)).reshape(co, -1
                                                       ).astype(jnp.bfloat16)


def _bn_affine(sums, sqs, gamma, beta, count, eps=_EPS):
    s = jnp.sum(sums, axis=0)
    sq = jnp.sum(sqs, axis=0)
    mean = s / count
    var = jnp.maximum(sq / count - mean * mean, 0.0)
    inv = lax.rsqrt(var + eps)
    scale = gamma.astype(jnp.float32).reshape(-1, 1) * inv
    shift = beta.astype(jnp.float32).reshape(-1, 1) - mean * scale
    return scale, shift


def kernel(x, w1, w2, g1, b1, g2, b2):
    B, C, H, W = x.shape
    xf = x.reshape(B, C, H * W).astype(jnp.float32)
    count = jnp.float32(B * H * W)

    w1p = _pack_w(w1)
    w2p = _pack_w(w2)
    ident_s = jnp.ones((C, 1), jnp.float32)
    ident_t = jnp.zeros((C, 1), jnp.float32)

    y1, s1, q1 = _conv_stage(xf, w1p, ident_s, ident_t,
                             bn_relu_in=False, H=H, W=W)
    scale1, shift1 = _bn_affine(s1, q1, g1, b1, count)

    y2, s2, q2 = _conv_stage(y1, w2p, scale1, shift1,
                             bn_relu_in=True, H=H, W=W)
    scale2, shift2 = _bn_affine(s2, q2, g2, b2, count)

    out = _bn_relu(y2, scale2, shift2, x.dtype)
    return out.reshape(B, C, H, W)


# speedup vs baseline: 1.0910x; 1.0910x over previous
import functools

import jax
import jax.numpy as jnp
from jax import lax
from jax.experimental import pallas as pl
from jax.experimental.pallas import tpu as pltpu

LANE = 128
_EPS = 1e-5
_G = 8


def _conv_kernel(x_ref, w_ref, scale_ref, shift_ref,
                 y_ref, sum_ref, sq_ref, xf_ref, pat_ref,
                 *, bn_relu_in, H, W, G):
    C = x_ref.shape[1]
    P = H * W
    PW = y_ref.shape[2]
    S = xf_ref.shape[1] // G
    base = W + 1

    xf_ref[...] = jnp.zeros_like(xf_ref)
    for i in range(G):
        x = x_ref[i][:, :P]
        if bn_relu_in:
            s = scale_ref[...]
            t = shift_ref[...]
            x = jnp.maximum(x.astype(jnp.float32) * s + t, 0.0)
        xf_ref[:, i * S + base:i * S + base + P] = x.astype(jnp.bfloat16)

    j = lax.rem(lax.broadcasted_iota(jnp.int32, (1, PW), 1), W)
    mask_l = (j != 0).astype(jnp.bfloat16)
    mask_r = (j != W - 1).astype(jnp.bfloat16)

    for i in range(G):
        for kh in range(3):
            for kw in range(3):
                t_idx = kh * 3 + kw
                b = i * S + base + (kh - 1) * W + (kw - 1)
                p = xf_ref[:, b:b + PW]
                if kw == 0:
                    p = p * mask_l
                elif kw == 2:
                    p = p * mask_r
                pat_ref[t_idx * C:(t_idx + 1) * C,
                        i * PW:(i + 1) * PW] = p

    acc = jnp.dot(w_ref[...], pat_ref[...],
                  preferred_element_type=jnp.float32)

    qq = lax.rem(lax.broadcasted_iota(jnp.int32, (1, G * PW), 1), PW)
    accm = acc * (qq < P).astype(jnp.float32)
    sum_ref[0] = jnp.sum(accm, axis=1, keepdims=True)
    sq_ref[0] = jnp.sum(accm * accm, axis=1, keepdims=True)

    for i in range(G):
        y_ref[i] = acc[:, i * PW:(i + 1) * PW].astype(y_ref.dtype)


def _bn_relu_kernel(y_ref, scale_ref, shift_ref, o_ref):
    G, C, P = o_ref.shape
    s = scale_ref[...]
    t = shift_ref[...]
    for i in range(G):
        o_ref[i] = jnp.maximum(y_ref[i][:, :P].astype(jnp.float32) * s + t,
                               0.0).astype(o_ref.dtype)


def _conv_stage(x, w_packed, in_scale, in_shift, *, bn_relu_in, H, W, PW):
    B, C, Pin = x.shape
    G = _G if B % _G == 0 else 1
    NG = B // G
    S = PW + 2 * (W + 1)

    kern = functools.partial(_conv_kernel, bn_relu_in=bn_relu_in,
                             H=H, W=W, G=G)
    grid_spec = pltpu.PrefetchScalarGridSpec(
        num_scalar_prefetch=0,
        grid=(NG,),
        in_specs=[
            pl.BlockSpec((G, C, Pin), lambda b: (b, 0, 0)),
            pl.BlockSpec((C, 9 * C), lambda b: (0, 0)),
            pl.BlockSpec((C, 1), lambda b: (0, 0)),
            pl.BlockSpec((C, 1), lambda b: (0, 0)),
        ],
        out_specs=[
            pl.BlockSpec((G, C, PW), lambda b: (b, 0, 0)),
            pl.BlockSpec((1, C, 1), lambda b: (b, 0, 0)),
            pl.BlockSpec((1, C, 1), lambda b: (b, 0, 0)),
        ],
        scratch_shapes=[
            pltpu.VMEM((C, G * S), jnp.bfloat16),
            pltpu.VMEM((9 * C, G * PW), jnp.bfloat16),
        ],
    )
    y, s, sq = pl.pallas_call(
        kern,
        out_shape=(
            jax.ShapeDtypeStruct((B, C, PW), jnp.bfloat16),
            jax.ShapeDtypeStruct((NG, C, 1), jnp.float32),
            jax.ShapeDtypeStruct((NG, C, 1), jnp.float32),
        ),
        grid_spec=grid_spec,
        compiler_params=pltpu.CompilerParams(
            dimension_semantics=("parallel",),
            vmem_limit_bytes=100 * 1024 * 1024,
        ),
    )(x, w_packed, in_scale, in_shift)
    return y, s, sq


def _bn_relu(y, scale, shift, out_dtype, P):
    B, C, PW = y.shape
    G = _G if B % _G == 0 else 1
    NG = B // G
    grid_spec = pltpu.PrefetchScalarGridSpec(
        num_scalar_prefetch=0,
        grid=(NG,),
        in_specs=[
            pl.BlockSpec((G, C, PW), lambda b: (b, 0, 0)),
            pl.BlockSpec((C, 1), lambda b: (0, 0)),
            pl.BlockSpec((C, 1), lambda b: (0, 0)),
        ],
        out_specs=pl.BlockSpec((G, C, P), lambda b: (b, 0, 0)),
    )
    return pl.pallas_call(
        _bn_relu_kernel,
        out_shape=jax.ShapeDtypeStruct((B, C, P), out_dtype),
        grid_spec=grid_spec,
        compiler_params=pltpu.CompilerParams(
            dimension_semantics=("parallel",),
            vmem_limit_bytes=100 * 1024 * 1024,
        ),
    )(y, scale, shift)


def _pack_w(w_hwio):
    co = w_hwio.shape[3]
    return jnp.transpose(w_hwio, (3, 0, 1, 2)).reshape(co, -1
                                                       ).astype(jnp.bfloat16)


def _bn_affine(sums, sqs, gamma, beta, count, eps=_EPS):
    s = jnp.sum(sums, axis=0)
    sq = jnp.sum(sqs, axis=0)
    mean = s / count
    var = jnp.maximum(sq / count - mean * mean, 0.0)
    inv = lax.rsqrt(var + eps)
    scale = gamma.astype(jnp.float32).reshape(-1, 1) * inv
    shift = beta.astype(jnp.float32).reshape(-1, 1) - mean * scale
    return scale, shift


def kernel(x, w1, w2, g1, b1, g2, b2):
    B, C, H, W = x.shape
    P = H * W
    PW = (P + LANE - 1) // LANE * LANE
    xf = x.reshape(B, C, P).astype(jnp.float32)
    count = jnp.float32(B * P)

    w1p = _pack_w(w1)
    w2p = _pack_w(w2)
    ident_s = jnp.ones((C, 1), jnp.float32)
    ident_t = jnp.zeros((C, 1), jnp.float32)

    y1, s1, q1 = _conv_stage(xf, w1p, ident_s, ident_t,
                             bn_relu_in=False, H=H, W=W, PW=PW)
    scale1, shift1 = _bn_affine(s1, q1, g1, b1, count)

    y2, s2, q2 = _conv_stage(y1, w2p, scale1, shift1,
                             bn_relu_in=True, H=H, W=W, PW=PW)
    scale2, shift2 = _bn_affine(s2, q2, g2, b2, count)

    out = _bn_relu(y2, scale2, shift2, x.dtype, P)
    return out.reshape(B, C, H, W)


# speedup vs baseline: 2.4257x; 2.2233x over previous
import functools

import jax
import jax.numpy as jnp
from jax import lax
from jax.experimental import pallas as pl
from jax.experimental.pallas import tpu as pltpu

_EPS = 1e-5


def _affine(sp_ref, qp_ref, g_ref, b_ref, count, eps=_EPS):
    s = sp_ref[0] + sp_ref[1]
    sq = qp_ref[0] + qp_ref[1]
    mean = s * (1.0 / count)
    var = jnp.maximum(sq * (1.0 / count) - mean * mean, 0.0)
    inv = lax.rsqrt(var + eps)
    scale = g_ref[...] * inv
    shift = b_ref[...] - mean * scale
    return scale, shift


def _conv_kernel(x_ref, w_ref, sp_ref, qp_ref, g_ref, b_ref,
                 y_ref, sum_ref, sq_ref, xpad_ref, lhs_ref,
                 *, bn_relu_in, H, W, count):
    P, Bh, C = x_ref.shape
    Wp = W + 2

    if bn_relu_in:
        scale, shift = _affine(sp_ref, qp_ref, g_ref, b_ref, count)
        s3 = scale.reshape(1, 1, C)
        t3 = shift.reshape(1, 1, C)

    wb = w_ref[...].astype(jnp.bfloat16)

    xpad_ref[...] = jnp.zeros_like(xpad_ref)
    for r in range(H):
        src = x_ref[r * W:(r + 1) * W]
        if bn_relu_in:
            src = jnp.maximum(src.astype(jnp.float32) * s3 + t3, 0.0)
        base = Wp + 2 + Wp * r
        xpad_ref[base:base + W] = src.astype(jnp.bfloat16)

    sums = jnp.zeros((1, C), jnp.float32)
    sqs = jnp.zeros((1, C), jnp.float32)
    for r in range(H):
        for kh in range(3):
            for kw in range(3):
                t_idx = kh * 3 + kw
                off = Wp * (kh - 1) + (kw - 1)
                a = Wp + 2 + Wp * r + off
                lhs_ref[:, t_idx * C:(t_idx + 1) * C] = (
                    xpad_ref[a:a + W].reshape(W * Bh, C))
        acc = jnp.dot(lhs_ref[...], wb,
                      preferred_element_type=jnp.float32)
        sums = sums + jnp.sum(acc, axis=0, keepdims=True)
        sqs = sqs + jnp.sum(acc * acc, axis=0, keepdims=True)
        y_ref[r * W:(r + 1) * W] = acc.reshape(W, Bh, C).astype(y_ref.dtype)

    sum_ref[0] = sums
    sq_ref[0] = sqs


def _bn_relu_kernel(y_ref, sp_ref, qp_ref, g_ref, b_ref, o_ref, *, count):
    P, Bh, C = o_ref.shape
    scale, shift = _affine(sp_ref, qp_ref, g_ref, b_ref, count)
    s3 = scale.reshape(1, 1, C)
    t3 = shift.reshape(1, 1, C)
    o_ref[...] = jnp.maximum(y_ref[...].astype(jnp.float32) * s3 + t3, 0.0
                             ).astype(o_ref.dtype)


def _conv_stage(x, w, sp, qp, g, b, *, bn_relu_in, H, W, count):
    P, B, C = x.shape
    NH = 2 if B % 16 == 0 else 1
    Bh = B // NH
    Wp = W + 2
    XP = Wp * (H + 2) + 8

    kern = functools.partial(_conv_kernel, bn_relu_in=bn_relu_in,
                             H=H, W=W, count=count)
    grid_spec = pltpu.PrefetchScalarGridSpec(
        num_scalar_prefetch=0,
        grid=(NH,),
        in_specs=[
            pl.BlockSpec((P, Bh, C), lambda h: (0, h, 0)),
            pl.BlockSpec((9 * C, C), lambda h: (0, 0)),
            pl.BlockSpec((2, 1, C), lambda h: (0, 0, 0)),
            pl.BlockSpec((2, 1, C), lambda h: (0, 0, 0)),
            pl.BlockSpec((1, C), lambda h: (0, 0)),
            pl.BlockSpec((1, C), lambda h: (0, 0)),
        ],
        out_specs=[
            pl.BlockSpec((P, Bh, C), lambda h: (0, h, 0)),
            pl.BlockSpec((1, 1, C), lambda h: (h, 0, 0)),
            pl.BlockSpec((1, 1, C), lambda h: (h, 0, 0)),
        ],
        scratch_shapes=[
            pltpu.VMEM((XP, Bh, C), jnp.bfloat16),
            pltpu.VMEM((W * Bh, 9 * C), jnp.bfloat16),
        ],
    )
    y, s, sq = pl.pallas_call(
        kern,
        out_shape=(
            jax.ShapeDtypeStruct((P, B, C), jnp.bfloat16),
            jax.ShapeDtypeStruct((2, 1, C), jnp.float32),
            jax.ShapeDtypeStruct((2, 1, C), jnp.float32),
        ),
        grid_spec=grid_spec,
        compiler_params=pltpu.CompilerParams(
            dimension_semantics=("parallel",),
            vmem_limit_bytes=100 * 1024 * 1024,
        ),
    )(x, w, sp, qp, g, b)
    return y, s, sq


def _bn_relu(y, sp, qp, g, b, out_dtype, *, count):
    P, B, C = y.shape
    NH = 2 if B % 16 == 0 else 1
    Bh = B // NH
    kern = functools.partial(_bn_relu_kernel, count=count)
    grid_spec = pltpu.PrefetchScalarGridSpec(
        num_scalar_prefetch=0,
        grid=(NH,),
        in_specs=[
            pl.BlockSpec((P, Bh, C), lambda h: (0, h, 0)),
            pl.BlockSpec((2, 1, C), lambda h: (0, 0, 0)),
            pl.BlockSpec((2, 1, C), lambda h: (0, 0, 0)),
            pl.BlockSpec((1, C), lambda h: (0, 0)),
            pl.BlockSpec((1, C), lambda h: (0, 0)),
        ],
        out_specs=pl.BlockSpec((P, Bh, C), lambda h: (0, h, 0)),
    )
    return pl.pallas_call(
        kern,
        out_shape=jax.ShapeDtypeStruct((P, B, C), out_dtype),
        grid_spec=grid_spec,
        compiler_params=pltpu.CompilerParams(
            dimension_semantics=("parallel",),
            vmem_limit_bytes=100 * 1024 * 1024,
        ),
    )(y, sp, qp, g, b)


def kernel(x, w1, w2, g1, b1, g2, b2):
    B, C, H, W = x.shape
    P = H * W
    count = float(B * P)

    xt = jnp.transpose(x.reshape(B, C, P), (2, 0, 1))

    w1r = w1.reshape(9 * C, C).astype(jnp.float32)
    w2r = w2.reshape(9 * C, C).astype(jnp.float32)
    g1r = g1.astype(jnp.float32).reshape(1, C)
    b1r = b1.astype(jnp.float32).reshape(1, C)
    g2r = g2.astype(jnp.float32).reshape(1, C)
    b2r = b2.astype(jnp.float32).reshape(1, C)
    zz = jnp.zeros((2, 1, C), jnp.float32)
    zo = jnp.ones((1, C), jnp.float32)

    y1, s1, q1 = _conv_stage(xt, w1r, zz, zz, zo, zo,
                             bn_relu_in=False, H=H, W=W, count=count)
    y2, s2, q2 = _conv_stage(y1, w2r, s1, q1, g1r, b1r,
                             bn_relu_in=True, H=H, W=W, count=count)
    out = _bn_relu(y2, s2, q2, g2r, b2r, x.dtype, count=count)

    return jnp.transpose(out, (1, 2, 0)).reshape(B, C, H, W)


# speedup vs baseline: 2.8441x; 1.1725x over previous
import functools

import jax
import jax.numpy as jnp
from jax import lax
from jax.experimental import pallas as pl
from jax.experimental.pallas import tpu as pltpu

_EPS = 1e-5


def _affine(sp_ref, qp_ref, g_ref, b_ref, count, eps=_EPS):
    s = sp_ref[0]
    sq = qp_ref[0]
    for i in range(1, sp_ref.shape[0]):
        s = s + sp_ref[i]
        sq = sq + qp_ref[i]
    mean = s * (1.0 / count)
    var = jnp.maximum(sq * (1.0 / count) - mean * mean, 0.0)
    inv = lax.rsqrt(var + eps)
    scale = g_ref[...] * inv
    shift = b_ref[...] - mean * scale
    return scale, shift


def _conv_kernel(x_ref, w_ref, sp_ref, qp_ref, g_ref, b_ref,
                 y_ref, sum_ref, sq_ref, xpad_ref, lhs_ref,
                 *, bn_relu_in, H, W, count):
    P, Bh, C = x_ref.shape
    Wp = W + 2

    if bn_relu_in:
        scale, shift = _affine(sp_ref, qp_ref, g_ref, b_ref, count)
        s3 = scale.reshape(1, 1, C)
        t3 = shift.reshape(1, 1, C)

    wb = w_ref[...].astype(jnp.bfloat16)

    xpad_ref[...] = jnp.zeros_like(xpad_ref)
    for r in range(H):
        src = x_ref[r * W:(r + 1) * W]
        if bn_relu_in:
            src = jnp.maximum(src.astype(jnp.float32) * s3 + t3, 0.0)
        base = Wp + 2 + Wp * r
        xpad_ref[base:base + W] = src.astype(jnp.bfloat16)

    sums = jnp.zeros((1, C), jnp.float32)
    sqs = jnp.zeros((1, C), jnp.float32)
    for r in range(H):
        for kh in range(3):
            for kw in range(3):
                t_idx = kh * 3 + kw
                off = Wp * (kh - 1) + (kw - 1)
                a = Wp + 2 + Wp * r + off
                lhs_ref[:, t_idx * C:(t_idx + 1) * C] = (
                    xpad_ref[a:a + W].reshape(W * Bh, C))
        acc = jnp.dot(lhs_ref[...], wb,
                      preferred_element_type=jnp.float32)
        sums = sums + jnp.sum(acc, axis=0, keepdims=True)
        sqs = sqs + jnp.sum(acc * acc, axis=0, keepdims=True)
        y_ref[r * W:(r + 1) * W] = acc.reshape(W, Bh, C).astype(y_ref.dtype)

    sum_ref[0] = sums
    sq_ref[0] = sqs


def _bn_relu_kernel(y_ref, sp_ref, qp_ref, g_ref, b_ref, o_ref, *, count):
    P, Bh, C = o_ref.shape
    scale, shift = _affine(sp_ref, qp_ref, g_ref, b_ref, count)
    s3 = scale.reshape(1, 1, C)
    t3 = shift.reshape(1, 1, C)
    o_ref[...] = jnp.maximum(y_ref[...].astype(jnp.float32) * s3 + t3, 0.0
                             ).astype(o_ref.dtype)


def _conv_stage(x, w, sp, qp, g, b, *, bn_relu_in, H, W, count):
    P, B, C = x.shape
    NH = B // 16 if B % 16 == 0 else 1
    Bh = B // NH
    Wp = W + 2
    XP = Wp * (H + 2) + 8

    kern = functools.partial(_conv_kernel, bn_relu_in=bn_relu_in,
                             H=H, W=W, count=count)
    grid_spec = pltpu.PrefetchScalarGridSpec(
        num_scalar_prefetch=0,
        grid=(NH,),
        in_specs=[
            pl.BlockSpec((P, Bh, C), lambda h: (0, h, 0)),
            pl.BlockSpec((9 * C, C), lambda h: (0, 0)),
            pl.BlockSpec((sp.shape[0], 1, C), lambda h: (0, 0, 0)),
            pl.BlockSpec((sp.shape[0], 1, C), lambda h: (0, 0, 0)),
            pl.BlockSpec((1, C), lambda h: (0, 0)),
            pl.BlockSpec((1, C), lambda h: (0, 0)),
        ],
        out_specs=[
            pl.BlockSpec((P, Bh, C), lambda h: (0, h, 0)),
            pl.BlockSpec((1, 1, C), lambda h: (h, 0, 0)),
            pl.BlockSpec((1, 1, C), lambda h: (h, 0, 0)),
        ],
        scratch_shapes=[
            pltpu.VMEM((XP, Bh, C), jnp.bfloat16),
            pltpu.VMEM((W * Bh, 9 * C), jnp.bfloat16),
        ],
    )
    y, s, sq = pl.pallas_call(
        kern,
        out_shape=(
            jax.ShapeDtypeStruct((P, B, C), jnp.bfloat16),
            jax.ShapeDtypeStruct((NH, 1, C), jnp.float32),
            jax.ShapeDtypeStruct((NH, 1, C), jnp.float32),
        ),
        grid_spec=grid_spec,
        compiler_params=pltpu.CompilerParams(
            dimension_semantics=("parallel",),
            vmem_limit_bytes=100 * 1024 * 1024,
        ),
    )(x, w, sp, qp, g, b)
    return y, s, sq


def _bn_relu(y, sp, qp, g, b, out_dtype, *, count):
    P, B, C = y.shape
    NH = B // 16 if B % 16 == 0 else 1
    Bh = B // NH
    kern = functools.partial(_bn_relu_kernel, count=count)
    grid_spec = pltpu.PrefetchScalarGridSpec(
        num_scalar_prefetch=0,
        grid=(NH,),
        in_specs=[
            pl.BlockSpec((P, Bh, C), lambda h: (0, h, 0)),
            pl.BlockSpec((sp.shape[0], 1, C), lambda h: (0, 0, 0)),
            pl.BlockSpec((sp.shape[0], 1, C), lambda h: (0, 0, 0)),
            pl.BlockSpec((1, C), lambda h: (0, 0)),
            pl.BlockSpec((1, C), lambda h: (0, 0)),
        ],
        out_specs=pl.BlockSpec((P, Bh, C), lambda h: (0, h, 0)),
    )
    return pl.pallas_call(
        kern,
        out_shape=jax.ShapeDtypeStruct((P, B, C), out_dtype),
        grid_spec=grid_spec,
        compiler_params=pltpu.CompilerParams(
            dimension_semantics=("parallel",),
            vmem_limit_bytes=100 * 1024 * 1024,
        ),
    )(y, sp, qp, g, b)


def kernel(x, w1, w2, g1, b1, g2, b2):
    B, C, H, W = x.shape
    P = H * W
    count = float(B * P)

    xt = jnp.transpose(x.reshape(B, C, P), (2, 0, 1))

    w1r = w1.reshape(9 * C, C).astype(jnp.float32)
    w2r = w2.reshape(9 * C, C).astype(jnp.float32)
    g1r = g1.astype(jnp.float32).reshape(1, C)
    b1r = b1.astype(jnp.float32).reshape(1, C)
    g2r = g2.astype(jnp.float32).reshape(1, C)
    b2r = b2.astype(jnp.float32).reshape(1, C)
    nparts = B // 16 if B % 16 == 0 else 1
    zz = jnp.zeros((nparts, 1, C), jnp.float32)
    zo = jnp.ones((1, C), jnp.float32)

    y1, s1, q1 = _conv_stage(xt, w1r, zz, zz, zo, zo,
                             bn_relu_in=False, H=H, W=W, count=count)
    y2, s2, q2 = _conv_stage(y1, w2r, s1, q1, g1r, b1r,
                             bn_relu_in=True, H=H, W=W, count=count)
    out = _bn_relu(y2, s2, q2, g2r, b2r, x.dtype, count=count)

    return jnp.transpose(out, (1, 2, 0)).reshape(B, C, H, W)


# speedup vs baseline: 2.9935x; 1.0525x over previous
import functools

import jax
import jax.numpy as jnp
from jax import lax
from jax.experimental import pallas as pl
from jax.experimental.pallas import tpu as pltpu

_EPS = 1e-5


def _affine(sp_ref, qp_ref, g_ref, b_ref, count, eps=_EPS):
    s = sp_ref[0]
    sq = qp_ref[0]
    for i in range(1, sp_ref.shape[0]):
        s = s + sp_ref[i]
        sq = sq + qp_ref[i]
    mean = s * (1.0 / count)
    var = jnp.maximum(sq * (1.0 / count) - mean * mean, 0.0)
    inv = lax.rsqrt(var + eps)
    scale = g_ref[...] * inv
    shift = b_ref[...] - mean * scale
    return scale, shift


def _conv_kernel(x_ref, w_ref, sp_ref, qp_ref, g_ref, b_ref,
                 y_ref, sum_ref, sq_ref, xpad_ref, lhs_ref,
                 *, bn_relu_in, H, W, count):
    P, Bh, C = x_ref.shape
    Wp = W + 2

    if bn_relu_in:
        scale, shift = _affine(sp_ref, qp_ref, g_ref, b_ref, count)
        s3 = scale.reshape(1, 1, C)
        t3 = shift.reshape(1, 1, C)

    wb = w_ref[...].astype(jnp.bfloat16)

    xpad_ref[...] = jnp.zeros_like(xpad_ref)
    for r in range(H):
        src = x_ref[r * W:(r + 1) * W]
        if bn_relu_in:
            src = jnp.maximum(src.astype(jnp.float32) * s3 + t3, 0.0)
        base = Wp + 2 + Wp * r
        xpad_ref[base:base + W] = src.astype(jnp.bfloat16)

    sums = jnp.zeros((1, C), jnp.float32)
    sqs = jnp.zeros((1, C), jnp.float32)
    for r in range(H):
        for kh in range(3):
            for kw in range(3):
                t_idx = kh * 3 + kw
                off = Wp * (kh - 1) + (kw - 1)
                a = Wp + 2 + Wp * r + off
                lhs_ref[:, t_idx * C:(t_idx + 1) * C] = (
                    xpad_ref[a:a + W].reshape(W * Bh, C))
        acc = jnp.dot(lhs_ref[...], wb,
                      preferred_element_type=jnp.float32)
        sums = sums + jnp.sum(acc, axis=0, keepdims=True)
        sqs = sqs + jnp.sum(acc * acc, axis=0, keepdims=True)
        y_ref[r * W:(r + 1) * W] = acc.reshape(W, Bh, C).astype(y_ref.dtype)

    sum_ref[0] = sums
    sq_ref[0] = sqs


def _bn_relu_kernel(y_ref, sp_ref, qp_ref, g_ref, b_ref, o_ref, *, count):
    P, Bh, C = o_ref.shape
    scale, shift = _affine(sp_ref, qp_ref, g_ref, b_ref, count)
    s3 = scale.reshape(1, 1, C)
    t3 = shift.reshape(1, 1, C)
    o_ref[...] = jnp.maximum(y_ref[...].astype(jnp.float32) * s3 + t3, 0.0
                             ).astype(o_ref.dtype)


def _conv_stage(x, w, sp, qp, g, b, *, bn_relu_in, H, W, count):
    P, B, C = x.shape
    NH = B // 8 if B % 8 == 0 else 1
    Bh = B // NH
    Wp = W + 2
    XP = Wp * (H + 2) + 8

    kern = functools.partial(_conv_kernel, bn_relu_in=bn_relu_in,
                             H=H, W=W, count=count)
    grid_spec = pltpu.PrefetchScalarGridSpec(
        num_scalar_prefetch=0,
        grid=(NH,),
        in_specs=[
            pl.BlockSpec((P, Bh, C), lambda h: (0, h, 0)),
            pl.BlockSpec((9 * C, C), lambda h: (0, 0)),
            pl.BlockSpec((sp.shape[0], 1, C), lambda h: (0, 0, 0)),
            pl.BlockSpec((sp.shape[0], 1, C), lambda h: (0, 0, 0)),
            pl.BlockSpec((1, C), lambda h: (0, 0)),
            pl.BlockSpec((1, C), lambda h: (0, 0)),
        ],
        out_specs=[
            pl.BlockSpec((P, Bh, C), lambda h: (0, h, 0)),
            pl.BlockSpec((1, 1, C), lambda h: (h, 0, 0)),
            pl.BlockSpec((1, 1, C), lambda h: (h, 0, 0)),
        ],
        scratch_shapes=[
            pltpu.VMEM((XP, Bh, C), jnp.bfloat16),
            pltpu.VMEM((W * Bh, 9 * C), jnp.bfloat16),
        ],
    )
    y, s, sq = pl.pallas_call(
        kern,
        out_shape=(
            jax.ShapeDtypeStruct((P, B, C), jnp.bfloat16),
            jax.ShapeDtypeStruct((NH, 1, C), jnp.float32),
            jax.ShapeDtypeStruct((NH, 1, C), jnp.float32),
        ),
        grid_spec=grid_spec,
        compiler_params=pltpu.CompilerParams(
            dimension_semantics=("parallel",),
            vmem_limit_bytes=100 * 1024 * 1024,
        ),
    )(x, w, sp, qp, g, b)
    return y, s, sq


def _bn_relu(y, sp, qp, g, b, out_dtype, *, count):
    P, B, C = y.shape
    NH = B // 8 if B % 8 == 0 else 1
    Bh = B // NH
    kern = functools.partial(_bn_relu_kernel, count=count)
    grid_spec = pltpu.PrefetchScalarGridSpec(
        num_scalar_prefetch=0,
        grid=(NH,),
        in_specs=[
            pl.BlockSpec((P, Bh, C), lambda h: (0, h, 0)),
            pl.BlockSpec((sp.shape[0], 1, C), lambda h: (0, 0, 0)),
            pl.BlockSpec((sp.shape[0], 1, C), lambda h: (0, 0, 0)),
            pl.BlockSpec((1, C), lambda h: (0, 0)),
            pl.BlockSpec((1, C), lambda h: (0, 0)),
        ],
        out_specs=pl.BlockSpec((P, Bh, C), lambda h: (0, h, 0)),
    )
    return pl.pallas_call(
        kern,
        out_shape=jax.ShapeDtypeStruct((P, B, C), out_dtype),
        grid_spec=grid_spec,
        compiler_params=pltpu.CompilerParams(
            dimension_semantics=("parallel",),
            vmem_limit_bytes=100 * 1024 * 1024,
        ),
    )(y, sp, qp, g, b)


def kernel(x, w1, w2, g1, b1, g2, b2):
    B, C, H, W = x.shape
    P = H * W
    count = float(B * P)

    xt = jnp.transpose(x.reshape(B, C, P), (2, 0, 1))

    w1r = w1.reshape(9 * C, C).astype(jnp.float32)
    w2r = w2.reshape(9 * C, C).astype(jnp.float32)
    g1r = g1.astype(jnp.float32).reshape(1, C)
    b1r = b1.astype(jnp.float32).reshape(1, C)
    g2r = g2.astype(jnp.float32).reshape(1, C)
    b2r = b2.astype(jnp.float32).reshape(1, C)
    nparts = B // 8 if B % 8 == 0 else 1
    zz = jnp.zeros((nparts, 1, C), jnp.float32)
    zo = jnp.ones((1, C), jnp.float32)

    y1, s1, q1 = _conv_stage(xt, w1r, zz, zz, zo, zo,
                             bn_relu_in=False, H=H, W=W, count=count)
    y2, s2, q2 = _conv_stage(y1, w2r, s1, q1, g1r, b1r,
                             bn_relu_in=True, H=H, W=W, count=count)
    out = _bn_relu(y2, s2, q2, g2r, b2r, x.dtype, count=count)

    return jnp.transpose(out, (1, 2, 0)).reshape(B, C, H, W)


# speedup vs baseline: 3.0845x; 1.0304x over previous
import functools

import jax
import jax.numpy as jnp
from jax import lax
from jax.experimental import pallas as pl
from jax.experimental.pallas import tpu as pltpu

_EPS = 1e-5


def _affine_from(s_ref, q_ref, g_ref, b_ref, count, eps=_EPS):
    mean = s_ref[...] * (1.0 / count)
    var = jnp.maximum(q_ref[...] * (1.0 / count) - mean * mean, 0.0)
    inv = lax.rsqrt(var + eps)
    scale = g_ref[...] * inv
    shift = b_ref[...] - mean * scale
    return scale, shift


def _fused_kernel(x_ref, w1_ref, w2_ref, g1_ref, b1_ref, g2_ref, b2_ref,
                  out_ref,
                  y1_ref, y2_ref, xpad_ref, lhs_ref, stg_ref,
                  s1_ref, q1_ref, s2_ref, q2_ref, sem,
                  *, H, W, count, NB):
    ph = pl.program_id(0)
    b = pl.program_id(1)
    P, Bh, C = x_ref.shape
    Wp = W + 2

    @pl.when((ph == 0) & (b == 0))
    def _init():
        s1_ref[...] = jnp.zeros_like(s1_ref)
        q1_ref[...] = jnp.zeros_like(q1_ref)
        s2_ref[...] = jnp.zeros_like(s2_ref)
        q2_ref[...] = jnp.zeros_like(q2_ref)

    def conv(src_rows, w_ref, dst_ref, s_ref, q_ref):
        wb = w_ref[...].astype(jnp.bfloat16)
        xpad_ref[...] = jnp.zeros_like(xpad_ref)
        for r in range(H):
            base = Wp + 2 + Wp * r
            xpad_ref[base:base + W] = src_rows(r)
        sums = jnp.zeros((1, C), jnp.float32)
        sqs = jnp.zeros((1, C), jnp.float32)
        for r in range(H):
            for kh in range(3):
                for kw in range(3):
                    t_idx = kh * 3 + kw
                    a = Wp + 2 + Wp * r + Wp * (kh - 1) + (kw - 1)
                    lhs_ref[:, t_idx * C:(t_idx + 1) * C] = (
                        xpad_ref[a:a + W].reshape(W * Bh, C))
            acc = jnp.dot(lhs_ref[...], wb,
                          preferred_element_type=jnp.float32)
            sums = sums + jnp.sum(acc, axis=0, keepdims=True)
            sqs = sqs + jnp.sum(acc * acc, axis=0, keepdims=True)
            dst_ref[r * W:(r + 1) * W] = acc.reshape(W, Bh, C).astype(
                jnp.bfloat16)
        s_ref[...] += sums
        q_ref[...] += sqs

    @pl.when(ph == 0)
    def _phase0():
        conv(lambda r: x_ref[r * W:(r + 1) * W].astype(jnp.bfloat16),
             w1_ref, y1_ref.at[b], s1_ref, q1_ref)

    @pl.when(ph == 1)
    def _phase1():
        scale, shift = _affine_from(s1_ref, q1_ref, g1_ref, b1_ref, count)
        s3 = scale.reshape(1, 1, C)
        t3 = shift.reshape(1, 1, C)
        src = y1_ref.at[b]

        def rows(r):
            v = src[r * W:(r + 1) * W].astype(jnp.float32)
            return jnp.maximum(v * s3 + t3, 0.0).astype(jnp.bfloat16)

        conv(rows, w2_ref, y2_ref.at[b], s2_ref, q2_ref)

    @pl.when(ph == 2)
    def _phase2():
        scale, shift = _affine_from(s2_ref, q2_ref, g2_ref, b2_ref, count)
        s3 = scale.reshape(1, 1, C)
        t3 = shift.reshape(1, 1, C)

        def dma(chunk):
            return pltpu.make_async_copy(
                stg_ref, out_ref.at[:, pl.ds(chunk * Bh, Bh), :], sem)

        @pl.when(b > 0)
        def _wait_prev():
            dma(b - 1).wait()

        v = y2_ref[b].astype(jnp.float32)
        stg_ref[...] = jnp.maximum(v * s3 + t3, 0.0)
        dma(b).start()

        @pl.when(b == NB - 1)
        def _wait_last():
            dma(b).wait()


def _residual_block(xt, w1r, w2r, g1r, b1r, g2r, b2r, *, H, W):
    P, B, C = xt.shape
    NB = B // 8 if B % 8 == 0 else 1
    Bh = B // NB
    count = float(B * P)
    Wp = W + 2
    XP = Wp * (H + 2) + 8

    kern = functools.partial(_fused_kernel, H=H, W=W, count=count, NB=NB)
    grid_spec = pltpu.PrefetchScalarGridSpec(
        num_scalar_prefetch=0,
        grid=(3, NB),
        in_specs=[
            pl.BlockSpec((P, Bh, C),
                         lambda ph, b: (0, jnp.where(ph == 0, b, 0), 0)),
            pl.BlockSpec((9 * C, C), lambda ph, b: (0, 0)),
            pl.BlockSpec((9 * C, C), lambda ph, b: (0, 0)),
            pl.BlockSpec((1, C), lambda ph, b: (0, 0)),
            pl.BlockSpec((1, C), lambda ph, b: (0, 0)),
            pl.BlockSpec((1, C), lambda ph, b: (0, 0)),
            pl.BlockSpec((1, C), lambda ph, b: (0, 0)),
        ],
        out_specs=pl.BlockSpec(memory_space=pltpu.MemorySpace.HBM),
        scratch_shapes=[
            pltpu.VMEM((NB, P, Bh, C), jnp.bfloat16),
            pltpu.VMEM((NB, P, Bh, C), jnp.bfloat16),
            pltpu.VMEM((XP, Bh, C), jnp.bfloat16),
            pltpu.VMEM((W * Bh, 9 * C), jnp.bfloat16),
            pltpu.VMEM((P, Bh, C), jnp.float32),
            pltpu.VMEM((1, C), jnp.float32),
            pltpu.VMEM((1, C), jnp.float32),
            pltpu.VMEM((1, C), jnp.float32),
            pltpu.VMEM((1, C), jnp.float32),
            pltpu.SemaphoreType.DMA,
        ],
    )
    return pl.pallas_call(
        kern,
        out_shape=jax.ShapeDtypeStruct((P, B, C), jnp.float32),
        grid_spec=grid_spec,
        compiler_params=pltpu.CompilerParams(
            dimension_semantics=("arbitrary", "arbitrary"),
            vmem_limit_bytes=100 * 1024 * 1024,
        ),
    )(xt, w1r, w2r, g1r, b1r, g2r, b2r)


def kernel(x, w1, w2, g1, b1, g2, b2):
    B, C, H, W = x.shape
    P = H * W

    xt = jnp.transpose(x.reshape(B, C, P), (2, 0, 1))

    w1r = w1.reshape(9 * C, C).astype(jnp.float32)
    w2r = w2.reshape(9 * C, C).astype(jnp.float32)
    g1r = g1.astype(jnp.float32).reshape(1, C)
    b1r = b1.astype(jnp.float32).reshape(1, C)
    g2r = g2.astype(jnp.float32).reshape(1, C)
    b2r = b2.astype(jnp.float32).reshape(1, C)

    out = _residual_block(xt, w1r, w2r, g1r, b1r, g2r, b2r, H=H, W=W)

    return jnp.transpose(out, (1, 2, 0)).reshape(B, C, H, W).astype(x.dtype)
